# baseline (speedup 1.0000x reference)
import jax
import jax.numpy as jnp
from jax.experimental import pallas as pl
from jax.experimental.pallas import tpu as pltpu


def _blend_body(a_ref, x_ref, y_ref, e_ref, o_ref):
    a = a_ref[...]
    m = x_ref[...] * y_ref[...]
    o_ref[...] = a + m * (e_ref[...] - a)


def kernel(arr, element, x_index, y_index):
    M, N, D = arr.shape
    at = arr.transpose(0, 2, 1)
    x3 = x_index.reshape(M, 1, 1)
    y3 = y_index.reshape(1, 1, N)
    e3 = element.reshape(1, D, 1)

    BM = min(256, M)
    BN = min(1024, N)
    grid = (M // BM, N // BN)

    out = pl.pallas_call(
        _blend_body,
        grid=grid,
        in_specs=[
            pl.BlockSpec((BM, D, BN), lambda i, j: (i, 0, j)),
            pl.BlockSpec((BM, 1, 1), lambda i, j: (i, 0, 0)),
            pl.BlockSpec((1, 1, BN), lambda i, j: (0, 0, j)),
            pl.BlockSpec((1, D, 1), lambda i, j: (0, 0, 0)),
        ],
        out_specs=pl.BlockSpec((BM, D, BN), lambda i, j: (i, 0, j)),
        out_shape=jax.ShapeDtypeStruct((M, D, N), jnp.float32),
        compiler_params=pltpu.CompilerParams(
            dimension_semantics=("parallel", "parallel"),
        ),
    )(at, x3, y3, e3)
    return out.transpose(0, 2, 1)


# speedup vs baseline: 1.0201x; 1.0201x over previous
import jax
import jax.numpy as jnp
from jax.experimental import pallas as pl
from jax.experimental.pallas import tpu as pltpu


def _blend_body(a_ref, x_ref, y_ref, e_ref, o_ref):
    a = a_ref[...]
    xt = jnp.transpose(x_ref[...], (2, 1, 0))
    m = xt * y_ref[...]
    o_ref[...] = a + m * (e_ref[...] - a)


def kernel(arr, element, x_index, y_index):
    M, N, D = arr.shape
    at = arr.transpose(0, 2, 1)
    BM = min(128, M)
    BN = min(2048, N)
    x3 = x_index.reshape(M // BM, 1, BM)
    y3 = y_index.reshape(1, 1, N)
    e3 = element.reshape(1, D, 1)
    grid = (M // BM, N // BN)

    out = pl.pallas_call(
        _blend_body,
        grid=grid,
        in_specs=[
            pl.BlockSpec((BM, D, BN), lambda i, j: (i, 0, j)),
            pl.BlockSpec((1, 1, BM), lambda i, j: (i, 0, 0)),
            pl.BlockSpec((1, 1, BN), lambda i, j: (0, 0, j)),
            pl.BlockSpec((1, D, 1), lambda i, j: (0, 0, 0)),
        ],
        out_specs=pl.BlockSpec((BM, D, BN), lambda i, j: (i, 0, j)),
        out_shape=jax.ShapeDtypeStruct((M, D, N), jnp.float32),
        compiler_params=pltpu.CompilerParams(
            dimension_semantics=("parallel", "parallel"),
        ),
    )(at, x3, y3, e3)
    return out.transpose(0, 2, 1)


# speedup vs baseline: 1.0240x; 1.0038x over previous
import jax
import jax.numpy as jnp
from jax.experimental import pallas as pl
from jax.experimental.pallas import tpu as pltpu


def _blend_body(a_ref, x_ref, y_ref, e_ref, o_ref):
    a = a_ref[...]
    xt = jnp.transpose(x_ref[...], (2, 1, 0))
    et = jnp.transpose(e_ref[...], (0, 2, 1))
    m = xt * y_ref[...]
    o_ref[...] = a + m * (et - a)


def kernel(arr, element, x_index, y_index):
    M, N, D = arr.shape
    at = arr.transpose(0, 2, 1)
    BM = min(128, M)
    BN = min(2048, N)
    x3 = x_index.reshape(M // BM, 1, BM)
    y3 = y_index.reshape(1, 1, N)
    e3 = element.reshape(1, 1, D)
    grid = (M // BM, N // BN)

    out = pl.pallas_call(
        _blend_body,
        grid=grid,
        in_specs=[
            pl.BlockSpec((BM, D, BN), lambda i, j: (i, 0, j)),
            pl.BlockSpec((1, 1, BM), lambda i, j: (i, 0, 0)),
            pl.BlockSpec((1, 1, BN), lambda i, j: (0, 0, j)),
            pl.BlockSpec((1, 1, D), lambda i, j: (0, 0, 0)),
        ],
        out_specs=pl.BlockSpec((BM, D, BN), lambda i, j: (i, 0, j)),
        out_shape=jax.ShapeDtypeStruct((M, D, N), jnp.float32),
        compiler_params=pltpu.CompilerParams(
            dimension_semantics=("parallel", "parallel"),
        ),
    )(at, x3, y3, e3)
    return out.transpose(0, 2, 1)
